# baseline (speedup 1.0000x reference)
import functools

import jax
import jax.numpy as jnp
from jax import lax
from jax.experimental import pallas as pl
from jax.experimental.pallas import tpu as pltpu
from jax.experimental.pallas import tpu_sc as plsc

_B = 4
_S = 2048
_D = 1024
_N = _B * _S
_NC = 2
_NS = 16
_NW = _NC * _NS
_PER_W = _N // _NW
_CHUNK = 16
_STEPS = _PER_W // _CHUNK
_NBUF = 3
_LANES = 16
_SPW = _S // _NW
_SCHUNKS = _SPW // _CHUNK


def _embed_kernel(ids_hbm, table_hbm, pos_hbm, out_hbm,
                  idx_v, pos_v, rows_v, semg, semp, sems):
    wid = lax.axis_index("s") * _NC + lax.axis_index("c")
    s_base = wid * _SPW
    pre_ds = [
        pltpu.async_copy(pos_hbm.at[pl.ds(s_base + k * _CHUNK, _CHUNK)],
                         pos_v.at[k], semp)
        for k in range(_SCHUNKS)]
    pltpu.sync_copy(ids_hbm.at[0, pl.ds(s_base, _SPW)],
                    idx_v.at[pl.ds(0, _SPW)])
    for b in range(1, _B):
        pre_ds.append(
            pltpu.async_copy(ids_hbm.at[b, pl.ds(s_base, _SPW)],
                             idx_v.at[pl.ds(b * _SPW, _SPW)], semp))

    def issue(g):
        p = lax.rem(g, _NBUF)
        idxs = idx_v[pl.ds(g * _CHUNK, _CHUNK)]
        pltpu.async_copy(table_hbm.at[idxs], rows_v.at[p], semg.at[p])

    def wait_store(j):
        p = lax.rem(j, _NBUF)
        pltpu.make_async_copy(
            rows_v.at[p], out_hbm.at[0, pl.ds(0, _CHUNK)], sems.at[p]).wait()

    def process(j):
        p = lax.rem(j, _NBUF)
        q = lax.rem(j, _SCHUNKS)
        pltpu.make_async_copy(
            table_hbm.at[pl.ds(0, _CHUNK)], rows_v.at[p], semg.at[p]).wait()

        @plsc.parallel_loop(0, _CHUNK, unroll=2)
        def _add_row(i):
            for c in range(_D // _LANES):
                sl = pl.ds(c * _LANES, _LANES)
                plsc.addupdate(rows_v.at[p, i, sl], pos_v[q, i, sl])

        b_idx = j // _SCHUNKS
        pltpu.async_copy(
            rows_v.at[p],
            out_hbm.at[b_idx, pl.ds(s_base + q * _CHUNK, _CHUNK)],
            sems.at[p])

    def body(g, carry):
        @pl.when(g >= _NBUF)
        def _():
            wait_store(g - _NBUF)

        issue(g)
        process(g - 1)
        return carry

    issue(0)
    for d in pre_ds:
        d.wait()
    lax.fori_loop(1, _STEPS, body, None)
    process(_STEPS - 1)
    for j in range(_STEPS - _NBUF, _STEPS):
        wait_store(j)


def kernel(input_ids, token_table, pos_table):
    mesh = plsc.VectorSubcoreMesh(core_axis_name="c", subcore_axis_name="s")
    run = functools.partial(
        pl.kernel,
        out_type=jax.ShapeDtypeStruct((_B, _S, _D), jnp.float32),
        mesh=mesh,
        scratch_types=[
            pltpu.VMEM((_PER_W,), jnp.int32),
            pltpu.VMEM((_SCHUNKS, _CHUNK, _D), jnp.float32),
            pltpu.VMEM((_NBUF, _CHUNK, _D), jnp.float32),
            pltpu.SemaphoreType.DMA((_NBUF,)),
            pltpu.SemaphoreType.DMA,
            pltpu.SemaphoreType.DMA((_NBUF,)),
        ],
    )(_embed_kernel)
    return run(input_ids.astype(jnp.int32), token_table, pos_table)


# speedup vs baseline: 1.0098x; 1.0098x over previous
import functools

import jax
import jax.numpy as jnp
from jax import lax
from jax.experimental import pallas as pl
from jax.experimental.pallas import tpu as pltpu
from jax.experimental.pallas import tpu_sc as plsc

_B = 4
_S = 2048
_D = 1024
_N = _B * _S
_NC = 2
_NS = 16
_NW = _NC * _NS
_PER_W = _N // _NW
_CHUNK = 16
_STEPS = _PER_W // _CHUNK
_NBUF = 3
_LANES = 16
_SPW = _S // _NW
_SCHUNKS = _SPW // _CHUNK


def _embed_kernel(ids_hbm, table_hbm, pos_hbm, out_hbm,
                  idx_v, pos_v, rows_v, semg, semp, sems):
    wid = lax.axis_index("s") * _NC + lax.axis_index("c")
    s_base = wid * _SPW
    pos_ds = [
        pltpu.async_copy(pos_hbm.at[pl.ds(s_base + k * _CHUNK, _CHUNK)],
                         pos_v.at[k], semp)
        for k in range(_SCHUNKS)]
    for b in range(_B):
        pltpu.sync_copy(ids_hbm.at[b, pl.ds(s_base, _SPW)],
                        idx_v.at[pl.ds(b * _SPW, _SPW)])

    def issue(g):
        p = lax.rem(g, _NBUF)
        idxs = idx_v[pl.ds(g * _CHUNK, _CHUNK)]
        pltpu.async_copy(table_hbm.at[idxs], rows_v.at[p], semg.at[p])

    def wait_store(j):
        p = lax.rem(j, _NBUF)
        pltpu.make_async_copy(
            rows_v.at[p], out_hbm.at[0, pl.ds(0, _CHUNK)], sems.at[p]).wait()

    def process(j):
        p = lax.rem(j, _NBUF)
        q = lax.rem(j, _SCHUNKS)
        pltpu.make_async_copy(
            table_hbm.at[pl.ds(0, _CHUNK)], rows_v.at[p], semg.at[p]).wait()

        @plsc.parallel_loop(0, _CHUNK)
        def _add_row(i):
            for c in range(_D // _LANES):
                sl = pl.ds(c * _LANES, _LANES)
                plsc.addupdate(rows_v.at[p, i, sl], pos_v[q, i, sl])

        b_idx = j // _SCHUNKS
        pltpu.async_copy(
            rows_v.at[p],
            out_hbm.at[b_idx, pl.ds(s_base + q * _CHUNK, _CHUNK)],
            sems.at[p])

    def body(g, carry):
        @pl.when(g >= _NBUF)
        def _():
            wait_store(g - _NBUF)

        issue(g)
        process(g - 1)
        return carry

    issue(0)
    for d in pos_ds:
        d.wait()
    lax.fori_loop(1, _STEPS, body, None)
    process(_STEPS - 1)
    for j in range(_STEPS - _NBUF, _STEPS):
        wait_store(j)


def kernel(input_ids, token_table, pos_table):
    mesh = plsc.VectorSubcoreMesh(core_axis_name="c", subcore_axis_name="s")
    run = functools.partial(
        pl.kernel,
        out_type=jax.ShapeDtypeStruct((_B, _S, _D), jnp.float32),
        mesh=mesh,
        scratch_types=[
            pltpu.VMEM((_PER_W,), jnp.int32),
            pltpu.VMEM((_SCHUNKS, _CHUNK, _D), jnp.float32),
            pltpu.VMEM((_NBUF, _CHUNK, _D), jnp.float32),
            pltpu.SemaphoreType.DMA((_NBUF,)),
            pltpu.SemaphoreType.DMA,
            pltpu.SemaphoreType.DMA((_NBUF,)),
        ],
    )(_embed_kernel)
    return run(input_ids.astype(jnp.int32), token_table, pos_table)
